# baseline (speedup 1.0000x reference)
import jax
import jax.numpy as jnp
from jax.experimental import pallas as pl
from jax.experimental.pallas import tpu as pltpu

_B = 16384
_H = 300
_BB = 4096
_NB = _B // _BB


def _body(x_ref, wz_ref, g0_ref, g1_ref, b0_ref, b1_ref, out_ref):
    x = x_ref[...]
    w = wz_ref[...]
    z = jnp.dot(w, x, preferred_element_type=jnp.float32)
    a0 = jnp.maximum(z[0:1, :] + b0_ref[...], 0.0)
    a1 = jnp.maximum(z[1:2, :] + b1_ref[...], 0.0)
    mx = jnp.maximum(a0, a1)
    e0 = jnp.exp(a0 - mx)
    e1 = jnp.exp(a1 - mx)
    lse = jnp.log(e0 + e1)
    s0 = (a0 - mx) - lse + g0_ref[...]
    s1 = (a1 - mx) - lse + g1_ref[...]
    mx2 = jnp.maximum(s0, s1)
    u0 = jnp.exp(s0 - mx2)
    u1 = jnp.exp(s1 - mx2)
    den = u0 + u1
    m = (u1 / den) > (u0 / den)
    mf0 = jnp.where(m, 0.0, 1.0)
    mf1 = jnp.where(m, 1.0, 0.0)
    x3 = x.reshape(_H, 2, _BB)
    out_ref[0] = x3[:, 0, :] * mf0
    out_ref[1] = x3[:, 1, :] * mf1


def kernel(se, n_tasks, W, b):
    del n_tasks
    seT = jnp.transpose(se, (2, 0, 1)).reshape(2 * _H, _B)
    eps = 1e-20
    u = jax.random.uniform(jax.random.key(1234), (_B, 2), dtype=jnp.float32)
    g = -jnp.log(-jnp.log(u + eps) + eps)
    g0 = g[:, 0].reshape(1, _B)
    g1 = g[:, 1].reshape(1, _B)
    wz = W.reshape(2, 2, _H).transpose(2, 1, 0).reshape(2 * _H, 2).T
    wz = wz.astype(jnp.bfloat16).astype(jnp.float32)

    out3 = pl.pallas_call(
        _body,
        grid=(_NB,),
        in_specs=[
            pl.BlockSpec((2 * _H, _BB), lambda i: (0, i)),
            pl.BlockSpec((2, 2 * _H), lambda i: (0, 0)),
            pl.BlockSpec((1, _BB), lambda i: (0, i)),
            pl.BlockSpec((1, _BB), lambda i: (0, i)),
            pl.BlockSpec((1, 1), lambda i: (0, 0)),
            pl.BlockSpec((1, 1), lambda i: (0, 0)),
        ],
        out_specs=pl.BlockSpec((2, _H, _BB), lambda i: (0, 0, i)),
        out_shape=jax.ShapeDtypeStruct((2, _H, _B), jnp.float32),
        compiler_params=pltpu.CompilerParams(
            dimension_semantics=("parallel",)),
    )(seT, wz, g0, g1, b[0].reshape(1, 1), b[1].reshape(1, 1))

    return out3.reshape(2 * _H, _B).T
